# baseline (speedup 1.0000x reference)
import functools

import jax
import jax.numpy as jnp
from jax import lax
from jax.experimental import pallas as pl
from jax.experimental.pallas import tpu as pltpu
from jax.experimental.pallas import tpu_sc as plsc

_ROWS = 64
_N = 8192
_LANES = 16
_VREGS = _N // _LANES
_NC = 2
_NS = 16
_NW = _NC * _NS
_ROWS_PER_W = _ROWS // _NW
_RADIX_BITS = 8
_RADIX = 1 << _RADIX_BITS
_PASSES = 4
_MIN32 = jnp.int32(-0x80000000)


def _sc_sort_rows(xm):
    mesh = plsc.VectorSubcoreMesh(core_axis_name="c", subcore_axis_name="s")

    vmem_k = pltpu.VMEM((_N,), jnp.float32)
    vmem_i = pltpu.VMEM((_N,), jnp.int32)
    vmem_h = pltpu.VMEM((_RADIX,), jnp.int32)

    @functools.partial(
        pl.kernel,
        out_type=[
            jax.ShapeDtypeStruct((_ROWS, _N), jnp.float32),
            jax.ShapeDtypeStruct((_ROWS, _N), jnp.int32),
        ],
        mesh=mesh,
        compiler_params=pltpu.CompilerParams(needs_layout_passes=False),
        scratch_types=[
            vmem_k, vmem_k, vmem_i, vmem_i,
            vmem_k, vmem_k, vmem_i, vmem_i,
            vmem_h, vmem_h,
            vmem_h, vmem_h,
        ],
    )
    def sort_kernel(x_hbm, vals_hbm, idx_hbm,
                    kaA, kbA, iaA, ibA, kaB, kbB, iaB, ibB,
                    h0A, h1A, h0B, h1B):
        wid = lax.axis_index("s") * _NC + lax.axis_index("c")
        lane_iota = lax.iota(jnp.int32, _LANES)
        row_a = wid * _ROWS_PER_W
        row_b = row_a + 1

        pltpu.sync_copy(x_hbm.at[row_a], kaA)
        pltpu.sync_copy(x_hbm.at[row_b], kaB)

        def zero_hists(ha, hb):
            zeros = jnp.zeros((_LANES,), jnp.int32)
            for j in range(_RADIX // _LANES):
                ha[pl.ds(j * _LANES, _LANES)] = zeros
                hb[pl.ds(j * _LANES, _LANES)] = zeros

        def hists_to_offsets(ha, hb):
            def off_body(j, running):
                ra, rb = running
                sl = pl.ds(j * _LANES, _LANES)
                va = ha[sl]
                vb = hb[sl]
                inca = plsc.cumsum(va)
                incb = plsc.cumsum(vb)
                ha[sl] = inca - va + ra
                hb[sl] = incb - vb + rb
                return ra + jnp.sum(va), rb + jnp.sum(vb)

            lax.fori_loop(0, _RADIX // _LANES, off_body,
                          (jnp.int32(0), jnp.int32(0)))

        zero_hists(h0A, h0B)

        def pro_body(i, _):
            sl = pl.ds(i * _LANES, _LANES)
            for ka, h0 in ((kaA, h0A), (kaB, h0B)):
                b = plsc.bitcast(ka[sl], jnp.int32)
                u = b ^ ((b >> 31) | _MIN32)
                ka[sl] = plsc.bitcast(u, jnp.float32)
                d = u & (_RADIX - 1)
                cnt, last_m = plsc.scan_count(d)
                plsc.addupdate_scatter(h0, [d], cnt, mask=last_m)
            return 0

        lax.fori_loop(0, _VREGS, pro_body, 0)

        bufsA = [(kaA, iaA), (kbA, ibA)]
        bufsB = [(kaB, iaB), (kbB, ibB)]
        histsA = [h0A, h1A]
        histsB = [h0B, h1B]
        for p in range(_PASSES):
            shift = p * _RADIX_BITS
            last_pass = p == _PASSES - 1
            first_pass = p == 0
            hA, hB = histsA[p % 2], histsB[p % 2]
            hA_nxt, hB_nxt = histsA[(p + 1) % 2], histsB[(p + 1) % 2]

            hists_to_offsets(hA, hB)
            if not last_pass:
                zero_hists(hA_nxt, hB_nxt)

            def perm_body(i, _, p=p, shift=shift, last_pass=last_pass,
                          first_pass=first_pass, hA=hA, hB=hB,
                          hA_nxt=hA_nxt, hB_nxt=hB_nxt):
                sl = pl.ds(i * _LANES, _LANES)
                rows = (
                    (bufsA[p % 2], bufsA[(p + 1) % 2], hA, hA_nxt),
                    (bufsB[p % 2], bufsB[(p + 1) % 2], hB, hB_nxt),
                )
                for (k_in, i_in), (k_out, i_out), hist, hist_nxt in rows:
                    u = plsc.bitcast(k_in[sl], jnp.int32)
                    if first_pass:
                        ix = i * _LANES + lane_iota
                    else:
                        ix = i_in[sl]
                    d = lax.shift_right_logical(u, shift) & (_RADIX - 1)
                    cnt, last_m = plsc.scan_count(d)
                    base = plsc.load_gather(hist, [d])
                    dest = base + cnt - 1
                    if last_pass:
                        out_bits = u ^ (jnp.invert(u >> 31) | _MIN32)
                        store = plsc.bitcast(out_bits, jnp.float32)
                    else:
                        store = plsc.bitcast(u, jnp.float32)
                    plsc.store_scatter(k_out, [dest], store)
                    plsc.store_scatter(i_out, [dest], ix)
                    plsc.addupdate_scatter(hist, [d], cnt, mask=last_m)
                    if not last_pass:
                        d2 = lax.shift_right_logical(
                            u, shift + _RADIX_BITS) & (_RADIX - 1)
                        cnt2, last2 = plsc.scan_count(d2)
                        plsc.addupdate_scatter(hist_nxt, [d2], cnt2,
                                               mask=last2)
                return 0

            lax.fori_loop(0, _VREGS, perm_body, 0)

        pltpu.sync_copy(kaA, vals_hbm.at[row_a])
        pltpu.sync_copy(iaA, idx_hbm.at[row_a])
        pltpu.sync_copy(kaB, vals_hbm.at[row_b])
        pltpu.sync_copy(iaB, idx_hbm.at[row_b])

    return sort_kernel(xm)


def kernel(x, stable, dim, descending, values, indices):
    del stable, dim, values, indices
    desc = jnp.asarray(descending)
    xm = jnp.where(desc, -x, x)
    vals_m, idx = _sc_sort_rows(xm)
    vals = jnp.where(desc, -vals_m, vals_m)
    return vals, idx.astype(jnp.int64)


# speedup vs baseline: 1.1521x; 1.1521x over previous
import functools

import jax
import jax.numpy as jnp
from jax import lax
from jax.experimental import pallas as pl
from jax.experimental.pallas import tpu as pltpu
from jax.experimental.pallas import tpu_sc as plsc

_ROWS = 64
_N = 8192
_LANES = 16
_VREGS = _N // _LANES
_NC = 2
_NS = 16
_NW = _NC * _NS
_ROWS_PER_W = _ROWS // _NW
_PASS_BITS = (11, 11, 10)
_PASS_SHIFTS = (0, 11, 22)
_HIST_WORDS = 1 << max(_PASS_BITS)
_PASSES = len(_PASS_BITS)
_MIN32 = jnp.int32(-0x80000000)


def _sc_sort_rows(xm):
    mesh = plsc.VectorSubcoreMesh(core_axis_name="c", subcore_axis_name="s")

    vmem_k = pltpu.VMEM((_N,), jnp.float32)
    vmem_i = pltpu.VMEM((_N,), jnp.int32)
    vmem_h = pltpu.VMEM((_HIST_WORDS,), jnp.int32)

    @functools.partial(
        pl.kernel,
        out_type=[
            jax.ShapeDtypeStruct((_ROWS, _N), jnp.float32),
            jax.ShapeDtypeStruct((_ROWS, _N), jnp.int32),
        ],
        mesh=mesh,
        compiler_params=pltpu.CompilerParams(needs_layout_passes=False),
        scratch_types=[
            vmem_k, vmem_k, vmem_i, vmem_i,
            vmem_k, vmem_k, vmem_i, vmem_i,
            vmem_h, vmem_h,
            vmem_h, vmem_h,
        ],
    )
    def sort_kernel(x_hbm, vals_hbm, idx_hbm,
                    kaA, kbA, iaA, ibA, kaB, kbB, iaB, ibB,
                    h0A, h1A, h0B, h1B):
        wid = lax.axis_index("s") * _NC + lax.axis_index("c")
        lane_iota = lax.iota(jnp.int32, _LANES)
        row_a = wid * _ROWS_PER_W
        row_b = row_a + 1

        pltpu.sync_copy(x_hbm.at[row_a], kaA)
        pltpu.sync_copy(x_hbm.at[row_b], kaB)

        def zero_hists(ha, hb, nbins):
            zeros = jnp.zeros((_LANES,), jnp.int32)

            def z_body(j, _):
                sl = pl.ds(j * _LANES, _LANES)
                ha[sl] = zeros
                hb[sl] = zeros
                return 0

            lax.fori_loop(0, nbins // _LANES, z_body, 0)

        def hists_to_offsets(ha, hb, nbins):
            def off_body(j, running):
                ra, rb = running
                sl = pl.ds(j * _LANES, _LANES)
                va = ha[sl]
                vb = hb[sl]
                inca = plsc.cumsum(va)
                incb = plsc.cumsum(vb)
                ha[sl] = inca - va + ra
                hb[sl] = incb - vb + rb
                return ra + jnp.sum(va), rb + jnp.sum(vb)

            lax.fori_loop(0, nbins // _LANES, off_body,
                          (jnp.int32(0), jnp.int32(0)))

        zero_hists(h0A, h0B, 1 << _PASS_BITS[0])

        def pro_body(i, _):
            sl = pl.ds(i * _LANES, _LANES)
            for ka, h0 in ((kaA, h0A), (kaB, h0B)):
                b = plsc.bitcast(ka[sl], jnp.int32)
                u = b ^ ((b >> 31) | _MIN32)
                ka[sl] = plsc.bitcast(u, jnp.float32)
                d = u & ((1 << _PASS_BITS[0]) - 1)
                cnt, last_m = plsc.scan_count(d)
                plsc.addupdate_scatter(h0, [d], cnt, mask=last_m)
            return 0

        lax.fori_loop(0, _VREGS, pro_body, 0)

        bufsA = [(kaA, iaA), (kbA, ibA)]
        bufsB = [(kaB, iaB), (kbB, ibB)]
        histsA = [h0A, h1A]
        histsB = [h0B, h1B]
        for p in range(_PASSES):
            shift = _PASS_SHIFTS[p]
            mask_p = (1 << _PASS_BITS[p]) - 1
            last_pass = p == _PASSES - 1
            first_pass = p == 0
            hA, hB = histsA[p % 2], histsB[p % 2]
            hA_nxt, hB_nxt = histsA[(p + 1) % 2], histsB[(p + 1) % 2]

            hists_to_offsets(hA, hB, 1 << _PASS_BITS[p])
            if not last_pass:
                zero_hists(hA_nxt, hB_nxt, 1 << _PASS_BITS[p + 1])
                shift_nxt = _PASS_SHIFTS[p + 1]
                mask_nxt = (1 << _PASS_BITS[p + 1]) - 1

            def perm_body(i, _, p=p, shift=shift, mask_p=mask_p,
                          last_pass=last_pass, first_pass=first_pass,
                          hA=hA, hB=hB, hA_nxt=hA_nxt, hB_nxt=hB_nxt):
                sl = pl.ds(i * _LANES, _LANES)
                rows = (
                    (bufsA[p % 2], bufsA[(p + 1) % 2], hA, hA_nxt),
                    (bufsB[p % 2], bufsB[(p + 1) % 2], hB, hB_nxt),
                )
                for (k_in, i_in), (k_out, i_out), hist, hist_nxt in rows:
                    u = plsc.bitcast(k_in[sl], jnp.int32)
                    if first_pass:
                        ix = i * _LANES + lane_iota
                    else:
                        ix = i_in[sl]
                    d = lax.shift_right_logical(u, shift) & mask_p
                    cnt, last_m = plsc.scan_count(d)
                    base = plsc.load_gather(hist, [d])
                    dest = base + cnt - 1
                    if last_pass:
                        out_bits = u ^ (jnp.invert(u >> 31) | _MIN32)
                        store = plsc.bitcast(out_bits, jnp.float32)
                    else:
                        store = plsc.bitcast(u, jnp.float32)
                    plsc.store_scatter(k_out, [dest], store)
                    plsc.store_scatter(i_out, [dest], ix)
                    plsc.addupdate_scatter(hist, [d], cnt, mask=last_m)
                    if not last_pass:
                        d2 = lax.shift_right_logical(u, shift_nxt) & mask_nxt
                        cnt2, last2 = plsc.scan_count(d2)
                        plsc.addupdate_scatter(hist_nxt, [d2], cnt2,
                                               mask=last2)
                return 0

            lax.fori_loop(0, _VREGS, perm_body, 0)

        out_k_A, out_i_A = bufsA[_PASSES % 2]
        out_k_B, out_i_B = bufsB[_PASSES % 2]
        pltpu.sync_copy(out_k_A, vals_hbm.at[row_a])
        pltpu.sync_copy(out_i_A, idx_hbm.at[row_a])
        pltpu.sync_copy(out_k_B, vals_hbm.at[row_b])
        pltpu.sync_copy(out_i_B, idx_hbm.at[row_b])

    return sort_kernel(xm)


def kernel(x, stable, dim, descending, values, indices):
    del stable, dim, values, indices
    desc = jnp.asarray(descending)
    xm = jnp.where(desc, -x, x)
    vals_m, idx = _sc_sort_rows(xm)
    vals = jnp.where(desc, -vals_m, vals_m)
    return vals, idx.astype(jnp.int64)


# speedup vs baseline: 1.1618x; 1.0084x over previous
import functools

import jax
import jax.numpy as jnp
from jax import lax
from jax.experimental import pallas as pl
from jax.experimental.pallas import tpu as pltpu
from jax.experimental.pallas import tpu_sc as plsc

_ROWS = 64
_N = 8192
_LANES = 16
_VREGS = _N // _LANES
_NC = 2
_NS = 16
_NW = _NC * _NS
_ROWS_PER_W = _ROWS // _NW
_PASS_BITS = (11, 11, 10)
_PASS_SHIFTS = (0, 11, 22)
_HIST_WORDS = 1 << max(_PASS_BITS)
_PASSES = len(_PASS_BITS)
_MIN32 = jnp.int32(-0x80000000)


def _sc_sort_rows(xm):
    mesh = plsc.VectorSubcoreMesh(core_axis_name="c", subcore_axis_name="s")

    vmem_k = pltpu.VMEM((_N,), jnp.float32)
    vmem_i = pltpu.VMEM((_N,), jnp.int32)
    vmem_h = pltpu.VMEM((_HIST_WORDS,), jnp.int32)

    @functools.partial(
        pl.kernel,
        out_type=[
            jax.ShapeDtypeStruct((_ROWS, _N), jnp.float32),
            jax.ShapeDtypeStruct((_ROWS, _N), jnp.int32),
        ],
        mesh=mesh,
        compiler_params=pltpu.CompilerParams(needs_layout_passes=False),
        scratch_types=[
            vmem_k, vmem_k, vmem_i, vmem_i,
            vmem_k, vmem_k, vmem_i, vmem_i,
            vmem_h, vmem_h,
            vmem_h, vmem_h,
        ],
    )
    def sort_kernel(x_hbm, vals_hbm, idx_hbm,
                    kaA, kbA, iaA, ibA, kaB, kbB, iaB, ibB,
                    h0A, h1A, h0B, h1B):
        wid = lax.axis_index("s") * _NC + lax.axis_index("c")
        lane_iota = lax.iota(jnp.int32, _LANES)
        row_a = wid * _ROWS_PER_W
        row_b = row_a + 1

        pltpu.sync_copy(x_hbm.at[row_a], kaA)
        pltpu.sync_copy(x_hbm.at[row_b], kaB)

        def zero_hists(ha, hb, nbins):
            zeros = jnp.zeros((_LANES,), jnp.int32)

            def z_body(j, _):
                sl = pl.ds(j * _LANES, _LANES)
                ha[sl] = zeros
                hb[sl] = zeros
                return 0

            lax.fori_loop(0, nbins // _LANES, z_body, 0)

        def hists_to_offsets(ha, hb, nbins):
            def off_body(j, running):
                ra, rb = running
                sl = pl.ds(j * _LANES, _LANES)
                va = ha[sl]
                vb = hb[sl]
                inca = plsc.cumsum(va)
                incb = plsc.cumsum(vb)
                ha[sl] = inca - va + ra
                hb[sl] = incb - vb + rb
                return ra + jnp.sum(va), rb + jnp.sum(vb)

            lax.fori_loop(0, nbins // _LANES, off_body,
                          (jnp.int32(0), jnp.int32(0)))

        zero_hists(h0A, h0B, 1 << _PASS_BITS[0])

        def pro_body(i, _):
            sl = pl.ds(i * _LANES, _LANES)
            for ka, h0 in ((kaA, h0A), (kaB, h0B)):
                b = plsc.bitcast(ka[sl], jnp.int32)
                u = b ^ ((b >> 31) | _MIN32)
                ka[sl] = plsc.bitcast(u, jnp.float32)
                d = u & ((1 << _PASS_BITS[0]) - 1)
                cnt, last_m = plsc.scan_count(d)
                plsc.addupdate_scatter(h0, [d], cnt, mask=last_m)
            return 0

        lax.fori_loop(0, _VREGS, pro_body, 0, unroll=4)

        bufsA = [(kaA, iaA), (kbA, ibA)]
        bufsB = [(kaB, iaB), (kbB, ibB)]
        histsA = [h0A, h1A]
        histsB = [h0B, h1B]
        for p in range(_PASSES):
            shift = _PASS_SHIFTS[p]
            mask_p = (1 << _PASS_BITS[p]) - 1
            last_pass = p == _PASSES - 1
            first_pass = p == 0
            hA, hB = histsA[p % 2], histsB[p % 2]
            hA_nxt, hB_nxt = histsA[(p + 1) % 2], histsB[(p + 1) % 2]

            hists_to_offsets(hA, hB, 1 << _PASS_BITS[p])
            if not last_pass:
                zero_hists(hA_nxt, hB_nxt, 1 << _PASS_BITS[p + 1])
                shift_nxt = _PASS_SHIFTS[p + 1]
                mask_nxt = (1 << _PASS_BITS[p + 1]) - 1

            def perm_body(i, _, p=p, shift=shift, mask_p=mask_p,
                          last_pass=last_pass, first_pass=first_pass,
                          hA=hA, hB=hB, hA_nxt=hA_nxt, hB_nxt=hB_nxt):
                sl = pl.ds(i * _LANES, _LANES)
                rows = (
                    (bufsA[p % 2], bufsA[(p + 1) % 2], hA, hA_nxt),
                    (bufsB[p % 2], bufsB[(p + 1) % 2], hB, hB_nxt),
                )
                for (k_in, i_in), (k_out, i_out), hist, hist_nxt in rows:
                    u = plsc.bitcast(k_in[sl], jnp.int32)
                    if first_pass:
                        ix = i * _LANES + lane_iota
                    else:
                        ix = i_in[sl]
                    d = lax.shift_right_logical(u, shift) & mask_p
                    cnt, last_m = plsc.scan_count(d)
                    base = plsc.load_gather(hist, [d])
                    dest = base + cnt - 1
                    if last_pass:
                        out_bits = u ^ (jnp.invert(u >> 31) | _MIN32)
                        store = plsc.bitcast(out_bits, jnp.float32)
                    else:
                        store = plsc.bitcast(u, jnp.float32)
                    plsc.store_scatter(k_out, [dest], store)
                    plsc.store_scatter(i_out, [dest], ix)
                    plsc.addupdate_scatter(hist, [d], cnt, mask=last_m)
                    if not last_pass:
                        d2 = lax.shift_right_logical(u, shift_nxt) & mask_nxt
                        cnt2, last2 = plsc.scan_count(d2)
                        plsc.addupdate_scatter(hist_nxt, [d2], cnt2,
                                               mask=last2)
                return 0

            lax.fori_loop(0, _VREGS, perm_body, 0, unroll=2)

        out_k_A, out_i_A = bufsA[_PASSES % 2]
        out_k_B, out_i_B = bufsB[_PASSES % 2]
        pltpu.sync_copy(out_k_A, vals_hbm.at[row_a])
        pltpu.sync_copy(out_i_A, idx_hbm.at[row_a])
        pltpu.sync_copy(out_k_B, vals_hbm.at[row_b])
        pltpu.sync_copy(out_i_B, idx_hbm.at[row_b])

    return sort_kernel(xm)


def kernel(x, stable, dim, descending, values, indices):
    del stable, dim, values, indices
    desc = jnp.asarray(descending)
    xm = jnp.where(desc, -x, x)
    vals_m, idx = _sc_sort_rows(xm)
    vals = jnp.where(desc, -vals_m, vals_m)
    return vals, idx.astype(jnp.int64)
